# baseline (speedup 1.0000x reference)
import jax
import jax.numpy as jnp
from jax.experimental import pallas as pl
from jax.experimental.pallas import tpu as pltpu

N = 10000
D = 128
H = 128
O = 128

NH = N // 2
BLOCK = 200
NBLK = NH // BLOCK
BLOCK2 = 1000
NBLK2 = 2 * (NH // BLOCK2)


def _fused_kernel(adjt_ref, adjb_ref, feats_ref, wa0t_ref, ba0_ref,
                  wa1t_ref, ba1_ref, wf0t_ref, bf0_ref, wf1t_ref, bf1_ref,
                  attk_ref, vvec_ref,
                  a_ref, h_ref, z_ref, att_ref,
                  a_s, h_s, colsum_s, kvec_s):
    i = pl.program_id(0)

    @pl.when(i < NBLK)
    def _encode():
        a2 = []
        for ref in (adjt_ref, adjb_ref):
            a1 = jax.lax.dot_general(ref[0], wa0t_ref[...],
                                     (((1,), (0,)), ((), ())),
                                     preferred_element_type=jnp.float32)
            a1 = jnp.maximum(a1 + ba0_ref[...], 0.0)
            a2.append(jax.lax.dot_general(a1, wa1t_ref[...],
                                          (((1,), (0,)), ((), ())),
                                          preferred_element_type=jnp.float32)
                      + ba1_ref[...])

        f2 = feats_ref[...].reshape(2 * BLOCK, D)
        h1 = jax.lax.dot_general(f2, wf0t_ref[...],
                                 (((1,), (0,)), ((), ())),
                                 preferred_element_type=jnp.float32)
        h1 = jnp.maximum(h1 + bf0_ref[...], 0.0)
        h2 = jax.lax.dot_general(h1, wf1t_ref[...],
                                 (((1,), (0,)), ((), ())),
                                 preferred_element_type=jnp.float32) + bf1_ref[...]

        a_pair = jnp.stack([a2[0], a2[1]], axis=0)
        h_pair = h2.reshape(2, BLOCK, O)
        a_ref[...] = a_pair
        h_ref[...] = h_pair
        a_s[:, pl.ds(i * BLOCK, BLOCK), :] = a_pair
        h_s[:, pl.ds(i * BLOCK, BLOCK), :] = h_pair

        part = jnp.sum((a_pair + h_pair).reshape(2 * BLOCK, O),
                       axis=0, keepdims=True)

        @pl.when(i == 0)
        def _():
            colsum_s[...] = part

        @pl.when(i > 0)
        def _():
            colsum_s[...] = colsum_s[...] + part

    @pl.when(i >= NBLK)
    def _finalize():
        @pl.when(i == NBLK)
        def _():
            kvec_s[...] = jax.lax.dot_general(
                colsum_s[...], attk_ref[...],
                (((1,), (0,)), ((), ())),
                preferred_element_type=jnp.float32) / N

        t = i - NBLK
        half = t // (NBLK2 // 2)
        jj = t % (NBLK2 // 2)
        k_row = kvec_s[...]
        a = a_s[half, pl.ds(jj * BLOCK2, BLOCK2), :]
        h = h_s[half, pl.ds(jj * BLOCK2, BLOCK2), :]
        la = jax.lax.dot_general(k_row, a, (((1,), (1,)), ((), ())),
                                 preferred_element_type=jnp.float32)
        lh = jax.lax.dot_general(k_row, h, (((1,), (1,)), ((), ())),
                                 preferred_element_type=jnp.float32)
        sa = jax.nn.sigmoid(la)
        sh = jax.nn.sigmoid(lh)

        v00 = vvec_ref[0, 0]
        v01 = vvec_ref[0, 1]
        v10 = vvec_ref[0, 2]
        v11 = vvec_ref[0, 3]
        tao = 2.0
        dt = (sa * (v00 - v01) + sh * (v10 - v11)) / tao
        att0_row = jax.nn.sigmoid(dt)
        att0 = att0_row.reshape(BLOCK2, 1)
        att1 = 1.0 - att0

        z_ref[...] = (h + att0 * (a - h))[None]
        att_ref[...] = jnp.concatenate([att0, att1], axis=1)[None]


def kernel(adj, feats, Wf0, bf0, Wf1, bf1, Wa0, ba0, Wa1, ba1,
           att_vec_k, att_vec_v):
    wa0t = Wa0.T
    wa1t = Wa1.T
    wf0t = Wf0.T
    wf1t = Wf1.T
    ba0r = ba0.reshape(1, H)
    ba1r = ba1.reshape(1, O)
    bf0r = bf0.reshape(1, H)
    bf1r = bf1.reshape(1, O)
    vvec = att_vec_v.reshape(1, 4)
    adj3 = adj.reshape(2, NH, N)
    feats3 = feats.reshape(2, NH, D)

    full = lambda shape: pl.BlockSpec(shape, lambda i: (0, 0))
    nb2h = NBLK2 // 2

    a3, h3, z3, att3 = pl.pallas_call(
        _fused_kernel,
        grid=(NBLK + NBLK2,),
        in_specs=[
            pl.BlockSpec((1, BLOCK, N),
                         lambda i: (0, jnp.minimum(i, NBLK - 1), 0)),
            pl.BlockSpec((1, BLOCK, N),
                         lambda i: (1, jnp.minimum(i, NBLK - 1), 0)),
            pl.BlockSpec((2, BLOCK, D),
                         lambda i: (0, jnp.minimum(i, NBLK - 1), 0)),
            full((N, H)),
            full((1, H)),
            full((H, O)),
            full((1, O)),
            full((D, H)),
            full((1, H)),
            full((H, O)),
            full((1, O)),
            full((O, O)),
            full((1, 4)),
        ],
        out_specs=[
            pl.BlockSpec((2, BLOCK, O),
                         lambda i: (0, jnp.minimum(i, NBLK - 1), 0)),
            pl.BlockSpec((2, BLOCK, O),
                         lambda i: (0, jnp.minimum(i, NBLK - 1), 0)),
            pl.BlockSpec((1, BLOCK2, O),
                         lambda i: ((jnp.maximum(i - NBLK, 0)) // nb2h,
                                    (jnp.maximum(i - NBLK, 0)) % nb2h, 0)),
            pl.BlockSpec((1, BLOCK2, 2),
                         lambda i: ((jnp.maximum(i - NBLK, 0)) // nb2h,
                                    (jnp.maximum(i - NBLK, 0)) % nb2h, 0)),
        ],
        out_shape=[
            jax.ShapeDtypeStruct((2, NH, O), jnp.float32),
            jax.ShapeDtypeStruct((2, NH, O), jnp.float32),
            jax.ShapeDtypeStruct((2, NH, O), jnp.float32),
            jax.ShapeDtypeStruct((2, NH, 2), jnp.float32),
        ],
        scratch_shapes=[
            pltpu.VMEM((2, NH, O), jnp.float32),
            pltpu.VMEM((2, NH, O), jnp.float32),
            pltpu.VMEM((1, O), jnp.float32),
            pltpu.VMEM((1, O), jnp.float32),
        ],
        compiler_params=pltpu.CompilerParams(
            dimension_semantics=("arbitrary",),
        ),
    )(adj3, adj3, feats3, wa0t, ba0r, wa1t, ba1r, wf0t, bf0r, wf1t, bf1r,
      att_vec_k, vvec)

    return (a3.reshape(N, O), h3.reshape(N, O), z3.reshape(N, O),
            att3.reshape(N, 2))


# speedup vs baseline: 1.0853x; 1.0853x over previous
import jax
import jax.numpy as jnp
from jax.experimental import pallas as pl
from jax.experimental.pallas import tpu as pltpu

N = 10000
D = 128
H = 128
O = 128

BLOCK = 400
NBLK = N // BLOCK
BLOCK2 = 2000
NBLK2 = N // BLOCK2
NSPLIT = 5
SUB = BLOCK // NSPLIT


def _fused_kernel(adj_hbm, feats_ref, wa0t_ref, ba0_ref, wa1t_ref, ba1_ref,
                  wf0t_ref, bf0_ref, wf1t_ref, bf1_ref, attk_ref, vvec_ref,
                  a_ref, h_ref, z_ref, att_ref,
                  adj_buf, sems, a_s, h_s, colsum_s, kvec_s):
    i = pl.program_id(0)

    def block_copies(blk, slot):
        return [
            pltpu.make_async_copy(
                adj_hbm.at[pl.ds(blk * BLOCK + k * SUB, SUB), :],
                adj_buf.at[slot, pl.ds(k * SUB, SUB), :],
                sems.at[slot, k])
            for k in range(NSPLIT)
        ]

    @pl.when(i == 0)
    def _prologue():
        for c in block_copies(0, 0):
            c.start()

    @pl.when(i + 1 < NBLK)
    def _prefetch():
        for c in block_copies(i + 1, (i + 1) % 2):
            c.start()

    @pl.when(i < NBLK)
    def _encode():
        slot = jax.lax.rem(i, 2)
        for c in block_copies(i, slot):
            c.wait()
        adj_blk = adj_buf[slot]

        a1 = jax.lax.dot_general(adj_blk, wa0t_ref[...],
                                 (((1,), (0,)), ((), ())),
                                 preferred_element_type=jnp.float32)
        a1 = jnp.maximum(a1 + ba0_ref[...], 0.0)
        a2 = jax.lax.dot_general(a1, wa1t_ref[...],
                                 (((1,), (0,)), ((), ())),
                                 preferred_element_type=jnp.float32) + ba1_ref[...]

        h1 = jax.lax.dot_general(feats_ref[...], wf0t_ref[...],
                                 (((1,), (0,)), ((), ())),
                                 preferred_element_type=jnp.float32)
        h1 = jnp.maximum(h1 + bf0_ref[...], 0.0)
        h2 = jax.lax.dot_general(h1, wf1t_ref[...],
                                 (((1,), (0,)), ((), ())),
                                 preferred_element_type=jnp.float32) + bf1_ref[...]

        a_ref[...] = a2
        h_ref[...] = h2
        a_s[pl.ds(i * BLOCK, BLOCK), :] = a2
        h_s[pl.ds(i * BLOCK, BLOCK), :] = h2

        part = jnp.sum(a2 + h2, axis=0, keepdims=True)

        @pl.when(i == 0)
        def _():
            colsum_s[...] = part

        @pl.when(i > 0)
        def _():
            colsum_s[...] = colsum_s[...] + part

    @pl.when(i >= NBLK)
    def _finalize():
        @pl.when(i == NBLK)
        def _():
            kvec_s[...] = jax.lax.dot_general(
                colsum_s[...], attk_ref[...],
                (((1,), (0,)), ((), ())),
                preferred_element_type=jnp.float32) / N

        j = i - NBLK
        k_row = kvec_s[...]
        a = a_s[pl.ds(j * BLOCK2, BLOCK2), :]
        h = h_s[pl.ds(j * BLOCK2, BLOCK2), :]
        la = jax.lax.dot_general(k_row, a, (((1,), (1,)), ((), ())),
                                 preferred_element_type=jnp.float32)
        lh = jax.lax.dot_general(k_row, h, (((1,), (1,)), ((), ())),
                                 preferred_element_type=jnp.float32)
        sa = jax.nn.sigmoid(la)
        sh = jax.nn.sigmoid(lh)

        v00 = vvec_ref[0, 0]
        v01 = vvec_ref[0, 1]
        v10 = vvec_ref[0, 2]
        v11 = vvec_ref[0, 3]
        tao = 2.0
        dt = (sa * (v00 - v01) + sh * (v10 - v11)) / tao
        att0_row = jax.nn.sigmoid(dt)
        att0 = att0_row.reshape(BLOCK2, 1)
        att1 = 1.0 - att0

        z_ref[...] = h + att0 * (a - h)
        att_ref[...] = jnp.concatenate([att0, att1], axis=1)


def kernel(adj, feats, Wf0, bf0, Wf1, bf1, Wa0, ba0, Wa1, ba1,
           att_vec_k, att_vec_v):
    wa0t = Wa0.T
    wa1t = Wa1.T
    wf0t = Wf0.T
    wf1t = Wf1.T
    ba0r = ba0.reshape(1, H)
    ba1r = ba1.reshape(1, O)
    bf0r = bf0.reshape(1, H)
    bf1r = bf1.reshape(1, O)
    vvec = att_vec_v.reshape(1, 4)

    full = lambda shape: pl.BlockSpec(shape, lambda i: (0, 0))
    rows1 = lambda shape: pl.BlockSpec(
        shape, lambda i: (jnp.minimum(i, NBLK - 1), 0))
    rows2 = lambda shape: pl.BlockSpec(
        shape, lambda i: (jnp.maximum(i - NBLK, 0), 0))

    a, h, z, att = pl.pallas_call(
        _fused_kernel,
        grid=(NBLK + NBLK2,),
        in_specs=[
            pl.BlockSpec(memory_space=pltpu.MemorySpace.HBM),
            rows1((BLOCK, D)),
            full((N, H)),
            full((1, H)),
            full((H, O)),
            full((1, O)),
            full((D, H)),
            full((1, H)),
            full((H, O)),
            full((1, O)),
            full((O, O)),
            full((1, 4)),
        ],
        out_specs=[
            rows1((BLOCK, O)),
            rows1((BLOCK, O)),
            rows2((BLOCK2, O)),
            rows2((BLOCK2, 2)),
        ],
        out_shape=[
            jax.ShapeDtypeStruct((N, O), jnp.float32),
            jax.ShapeDtypeStruct((N, O), jnp.float32),
            jax.ShapeDtypeStruct((N, O), jnp.float32),
            jax.ShapeDtypeStruct((N, 2), jnp.float32),
        ],
        scratch_shapes=[
            pltpu.VMEM((2, BLOCK, N), jnp.float32),
            pltpu.SemaphoreType.DMA((2, NSPLIT)),
            pltpu.VMEM((N, O), jnp.float32),
            pltpu.VMEM((N, O), jnp.float32),
            pltpu.VMEM((1, O), jnp.float32),
            pltpu.VMEM((1, O), jnp.float32),
        ],
        compiler_params=pltpu.CompilerParams(
            dimension_semantics=("arbitrary",),
        ),
    )(adj, feats, wa0t, ba0r, wa1t, ba1r, wf0t, bf0r, wf1t, bf1r,
      att_vec_k, vvec)

    return (a, h, z, att)


# speedup vs baseline: 1.0980x; 1.0118x over previous
import jax
import jax.numpy as jnp
from jax.experimental import pallas as pl
from jax.experimental.pallas import tpu as pltpu

N = 10000
D = 128
H = 128
O = 128

BLOCK = 400
NBLK = N // BLOCK
BLOCK2 = 2000
NBLK2 = N // BLOCK2


def _fused_kernel(adj_ref, feats_ref, wa0t_ref, ba0_ref, wa1t_ref, ba1_ref,
                  wf0t_ref, bf0_ref, wf1t_ref, bf1_ref, attk_ref, vvec_ref,
                  a_ref, h_ref, z_ref, att_ref,
                  a_s, h_s, colsum_s, kvec_s):
    i = pl.program_id(0)

    @pl.when(i < NBLK)
    def _encode():
        a1 = jax.lax.dot_general(adj_ref[...], wa0t_ref[...],
                                 (((1,), (0,)), ((), ())),
                                 preferred_element_type=jnp.float32)
        a1 = jnp.maximum(a1 + ba0_ref[...], 0.0)
        a2 = jax.lax.dot_general(a1, wa1t_ref[...],
                                 (((1,), (0,)), ((), ())),
                                 preferred_element_type=jnp.float32) + ba1_ref[...]

        h1 = jax.lax.dot_general(feats_ref[...], wf0t_ref[...],
                                 (((1,), (0,)), ((), ())),
                                 preferred_element_type=jnp.float32)
        h1 = jnp.maximum(h1 + bf0_ref[...], 0.0)
        h2 = jax.lax.dot_general(h1, wf1t_ref[...],
                                 (((1,), (0,)), ((), ())),
                                 preferred_element_type=jnp.float32) + bf1_ref[...]

        a_ref[...] = a2
        h_ref[...] = h2
        a_s[pl.ds(i * BLOCK, BLOCK), :] = a2
        h_s[pl.ds(i * BLOCK, BLOCK), :] = h2

        part = jnp.sum(a2 + h2, axis=0, keepdims=True)

        @pl.when(i == 0)
        def _():
            colsum_s[...] = part

        @pl.when(i > 0)
        def _():
            colsum_s[...] = colsum_s[...] + part

    @pl.when(i >= NBLK)
    def _finalize():
        @pl.when(i == NBLK)
        def _():
            kvec_s[...] = jax.lax.dot_general(
                colsum_s[...], attk_ref[...],
                (((1,), (0,)), ((), ())),
                preferred_element_type=jnp.float32) / N

        j = i - NBLK
        k_row = kvec_s[...]
        a = a_s[pl.ds(j * BLOCK2, BLOCK2), :]
        h = h_s[pl.ds(j * BLOCK2, BLOCK2), :]
        la = jax.lax.dot_general(k_row, a, (((1,), (1,)), ((), ())),
                                 preferred_element_type=jnp.float32)
        lh = jax.lax.dot_general(k_row, h, (((1,), (1,)), ((), ())),
                                 preferred_element_type=jnp.float32)
        sa = jax.nn.sigmoid(la)
        sh = jax.nn.sigmoid(lh)

        v00 = vvec_ref[0, 0]
        v01 = vvec_ref[0, 1]
        v10 = vvec_ref[0, 2]
        v11 = vvec_ref[0, 3]
        tao = 2.0
        dt = (sa * (v00 - v01) + sh * (v10 - v11)) / tao
        att0_row = jax.nn.sigmoid(dt)
        att0 = att0_row.reshape(BLOCK2, 1)
        att1 = 1.0 - att0

        z_ref[...] = h + att0 * (a - h)
        att_ref[...] = jnp.concatenate([att0, att1], axis=1)


def kernel(adj, feats, Wf0, bf0, Wf1, bf1, Wa0, ba0, Wa1, ba1,
           att_vec_k, att_vec_v):
    wa0t = Wa0.T
    wa1t = Wa1.T
    wf0t = Wf0.T
    wf1t = Wf1.T
    ba0r = ba0.reshape(1, H)
    ba1r = ba1.reshape(1, O)
    bf0r = bf0.reshape(1, H)
    bf1r = bf1.reshape(1, O)
    vvec = att_vec_v.reshape(1, 4)

    full = lambda shape: pl.BlockSpec(shape, lambda i: (0, 0))
    rows1 = lambda shape: pl.BlockSpec(
        shape, lambda i: (jnp.minimum(i, NBLK - 1), 0))
    rows2 = lambda shape: pl.BlockSpec(
        shape, lambda i: (jnp.maximum(i - NBLK, 0), 0))

    a, h, z, att = pl.pallas_call(
        _fused_kernel,
        grid=(NBLK + NBLK2,),
        in_specs=[
            rows1((BLOCK, N)),
            rows1((BLOCK, D)),
            full((N, H)),
            full((1, H)),
            full((H, O)),
            full((1, O)),
            full((D, H)),
            full((1, H)),
            full((H, O)),
            full((1, O)),
            full((O, O)),
            full((1, 4)),
        ],
        out_specs=[
            rows1((BLOCK, O)),
            rows1((BLOCK, O)),
            rows2((BLOCK2, O)),
            rows2((BLOCK2, 2)),
        ],
        out_shape=[
            jax.ShapeDtypeStruct((N, O), jnp.float32),
            jax.ShapeDtypeStruct((N, O), jnp.float32),
            jax.ShapeDtypeStruct((N, O), jnp.float32),
            jax.ShapeDtypeStruct((N, 2), jnp.float32),
        ],
        scratch_shapes=[
            pltpu.VMEM((N, O), jnp.float32),
            pltpu.VMEM((N, O), jnp.float32),
            pltpu.VMEM((1, O), jnp.float32),
            pltpu.VMEM((1, O), jnp.float32),
        ],
        compiler_params=pltpu.CompilerParams(
            dimension_semantics=("arbitrary",),
        ),
    )(adj, feats, wa0t, ba0r, wa1t, ba1r, wf0t, bf0r, wf1t, bf1r,
      att_vec_k, vvec)

    return (a, h, z, att)


# speedup vs baseline: 1.1051x; 1.0064x over previous
import jax
import jax.numpy as jnp
from jax.experimental import pallas as pl
from jax.experimental.pallas import tpu as pltpu

N = 10000
D = 128
H = 128
O = 128

BLOCK = 400
NBLK = N // BLOCK
BLOCK2 = 5000
NBLK2 = N // BLOCK2


def _fused_kernel(adj_ref, feats_ref, wa0t_ref, ba0_ref, wa1t_ref, ba1_ref,
                  wf0t_ref, bf0_ref, wf1t_ref, bf1_ref, attk_ref, vvec_ref,
                  a_ref, h_ref, z_ref, att_ref,
                  a_s, h_s, colsum_s, kvec_s):
    i = pl.program_id(0)

    @pl.when(i < NBLK)
    def _encode():
        a1 = jax.lax.dot_general(adj_ref[...], wa0t_ref[...],
                                 (((1,), (0,)), ((), ())),
                                 preferred_element_type=jnp.float32)
        a1 = jnp.maximum(a1 + ba0_ref[...], 0.0)
        a2 = jax.lax.dot_general(a1, wa1t_ref[...],
                                 (((1,), (0,)), ((), ())),
                                 preferred_element_type=jnp.float32) + ba1_ref[...]

        h1 = jax.lax.dot_general(feats_ref[...], wf0t_ref[...],
                                 (((1,), (0,)), ((), ())),
                                 preferred_element_type=jnp.float32)
        h1 = jnp.maximum(h1 + bf0_ref[...], 0.0)
        h2 = jax.lax.dot_general(h1, wf1t_ref[...],
                                 (((1,), (0,)), ((), ())),
                                 preferred_element_type=jnp.float32) + bf1_ref[...]

        a_ref[...] = a2
        h_ref[...] = h2
        a_s[pl.ds(i * BLOCK, BLOCK), :] = a2
        h_s[pl.ds(i * BLOCK, BLOCK), :] = h2

        part = jnp.sum(a2 + h2, axis=0, keepdims=True)

        @pl.when(i == 0)
        def _():
            colsum_s[...] = part

        @pl.when(i > 0)
        def _():
            colsum_s[...] = colsum_s[...] + part

    @pl.when(i >= NBLK)
    def _finalize():
        @pl.when(i == NBLK)
        def _():
            kvec_s[...] = jax.lax.dot_general(
                colsum_s[...], attk_ref[...],
                (((1,), (0,)), ((), ())),
                preferred_element_type=jnp.float32) / N

        j = i - NBLK
        k_row = kvec_s[...]
        a = a_s[pl.ds(j * BLOCK2, BLOCK2), :]
        h = h_s[pl.ds(j * BLOCK2, BLOCK2), :]
        la = jax.lax.dot_general(k_row, a, (((1,), (1,)), ((), ())),
                                 preferred_element_type=jnp.float32)
        lh = jax.lax.dot_general(k_row, h, (((1,), (1,)), ((), ())),
                                 preferred_element_type=jnp.float32)
        sa = jax.nn.sigmoid(la)
        sh = jax.nn.sigmoid(lh)

        v00 = vvec_ref[0, 0]
        v01 = vvec_ref[0, 1]
        v10 = vvec_ref[0, 2]
        v11 = vvec_ref[0, 3]
        tao = 2.0
        dt = (sa * (v00 - v01) + sh * (v10 - v11)) / tao
        att0_row = jax.nn.sigmoid(dt)
        att0 = att0_row.reshape(BLOCK2, 1)
        att1 = 1.0 - att0

        z_ref[...] = h + att0 * (a - h)
        att_ref[...] = jnp.concatenate([att0, att1], axis=1)


def kernel(adj, feats, Wf0, bf0, Wf1, bf1, Wa0, ba0, Wa1, ba1,
           att_vec_k, att_vec_v):
    wa0t = Wa0.T
    wa1t = Wa1.T
    wf0t = Wf0.T
    wf1t = Wf1.T
    ba0r = ba0.reshape(1, H)
    ba1r = ba1.reshape(1, O)
    bf0r = bf0.reshape(1, H)
    bf1r = bf1.reshape(1, O)
    vvec = att_vec_v.reshape(1, 4)

    full = lambda shape: pl.BlockSpec(shape, lambda i: (0, 0))
    rows1 = lambda shape: pl.BlockSpec(
        shape, lambda i: (jnp.minimum(i, NBLK - 1), 0))
    rows2 = lambda shape: pl.BlockSpec(
        shape, lambda i: (jnp.maximum(i - NBLK, 0), 0))

    a, h, z, att = pl.pallas_call(
        _fused_kernel,
        grid=(NBLK + NBLK2,),
        in_specs=[
            rows1((BLOCK, N)),
            rows1((BLOCK, D)),
            full((N, H)),
            full((1, H)),
            full((H, O)),
            full((1, O)),
            full((D, H)),
            full((1, H)),
            full((H, O)),
            full((1, O)),
            full((O, O)),
            full((1, 4)),
        ],
        out_specs=[
            rows1((BLOCK, O)),
            rows1((BLOCK, O)),
            rows2((BLOCK2, O)),
            rows2((BLOCK2, 2)),
        ],
        out_shape=[
            jax.ShapeDtypeStruct((N, O), jnp.float32),
            jax.ShapeDtypeStruct((N, O), jnp.float32),
            jax.ShapeDtypeStruct((N, O), jnp.float32),
            jax.ShapeDtypeStruct((N, 2), jnp.float32),
        ],
        scratch_shapes=[
            pltpu.VMEM((N, O), jnp.float32),
            pltpu.VMEM((N, O), jnp.float32),
            pltpu.VMEM((1, O), jnp.float32),
            pltpu.VMEM((1, O), jnp.float32),
        ],
        compiler_params=pltpu.CompilerParams(
            dimension_semantics=("arbitrary",),
        ),
    )(adj, feats, wa0t, ba0r, wa1t, ba1r, wf0t, bf0r, wf1t, bf1r,
      att_vec_k, vvec)

    return (a, h, z, att)
